# baseline (speedup 1.0000x reference)
import functools

import jax
import jax.numpy as jnp
from jax import lax
from jax.experimental import pallas as pl
from jax.experimental.pallas import tpu as pltpu
from jax.experimental.pallas import tpu_sc as plsc

N_CLASSES = 32
CCE_W = 1.0
DICE_W = 0.5
EPS = 1e-08
BN = 32768

_info = plsc.get_sparse_core_info()
_NC, _NS, _L = _info.num_cores, _info.num_subcores, _info.num_lanes
_NW = _NC * _NS
_NBINS = N_CLASSES * N_CLASSES
_CHUNK = 512


def _tc_kernel(pred_ref, gt_ref, w_ref, cce_ref, bins_ref, cce_acc):
    i = pl.program_id(0)
    nsteps = pl.num_programs(0)

    @pl.when(i == 0)
    def _init():
        cce_acc[0, 0] = 0.0

    x = pred_ref[...]
    g = gt_ref[...]
    wcol = w_ref[...]
    ones8 = jnp.ones((8, N_CLASSES), jnp.float32)

    e = jnp.exp(x)
    s = lax.dot_general(ones8, e, (((1,), (0,)), ((), ())),
                        preferred_element_type=jnp.float32)[0:1]
    q = e / s

    eq = jnp.exp(q)
    t = lax.dot_general(ones8, eq, (((1,), (0,)), ((), ())),
                        preferred_element_type=jnp.float32)[0:1]
    lse = jnp.log(t)
    gw = g * wcol
    sgw = lax.dot_general(ones8, gw, (((1,), (0,)), ((), ())),
                          preferred_element_type=jnp.float32)[0:1]
    cce_acc[0, 0] += jnp.sum(lse * sgw) - jnp.sum(gw * q)

    m = jnp.max(x, axis=0, keepdims=True)
    gm = jnp.max(g, axis=0, keepdims=True)
    pred_oh = (x == m).astype(jnp.float32)
    gt_oh = (g == gm).astype(jnp.float32)
    cls8 = jax.lax.broadcasted_iota(
        jnp.int32, (8, N_CLASSES), 1).astype(jnp.float32)
    pidx = lax.dot_general(cls8, pred_oh, (((1,), (0,)), ((), ())),
                           preferred_element_type=jnp.float32)[0:1]
    gidx = lax.dot_general(cls8, gt_oh, (((1,), (0,)), ((), ())),
                           preferred_element_type=jnp.float32)[0:1]
    binf = jnp.clip(gidx * N_CLASSES + pidx, 0.0, float(_NBINS - 1))
    bins_ref[...] = binf.astype(jnp.int32)

    @pl.when(i == nsteps - 1)
    def _finish():
        cce_ref[...] = jnp.full((1, 1), cce_acc[0, 0], dtype=jnp.float32)


def _make_sc_hist(n):
    per_w = n // _NW
    mesh = plsc.VectorSubcoreMesh(core_axis_name="c", subcore_axis_name="s")

    @functools.partial(
        pl.kernel, mesh=mesh,
        out_type=jax.ShapeDtypeStruct((_NC, _NBINS, _L), jnp.float32),
        scratch_types=[
            pltpu.VMEM((per_w,), jnp.int32),
            pltpu.VMEM((_CHUNK, _L), jnp.float32),
            pltpu.VMEM_SHARED((_NBINS, _L), jnp.float32),
        ],
    )
    def hist_k(bins_hbm, ones_hbm, zeros_hbm, out_hbm, idx_v, ones_v, shared):
        cid = lax.axis_index("c")
        sid = lax.axis_index("s")
        wid = sid * _NC + cid
        pltpu.sync_copy(bins_hbm.at[pl.ds(wid * per_w, per_w)], idx_v)
        pltpu.sync_copy(ones_hbm, ones_v)

        @pl.when(sid == 0)
        def _zero():
            pltpu.sync_copy(zeros_hbm, shared)
        plsc.subcore_barrier()
        for k in range(per_w // _CHUNK):
            pltpu.sync_copy(ones_v,
                            shared.at[idx_v.at[pl.ds(k * _CHUNK, _CHUNK)]],
                            add=True)
        plsc.subcore_barrier()

        @pl.when(sid == 0)
        def _out():
            pltpu.sync_copy(shared, out_hbm.at[cid])
    return hist_k


def _combine_kernel(parts_ref, cce_ref, w_ref, out_ref, *, n_total):
    p = parts_ref[0] + parts_ref[1]
    conf = p[:, 0:1].reshape(N_CLASSES, N_CLASSES)
    eye = (jax.lax.broadcasted_iota(jnp.int32, conf.shape, 0)
           == jax.lax.broadcasted_iota(jnp.int32, conf.shape, 1))
    tp = jnp.sum(jnp.where(eye, conf, 0.0), axis=0, keepdims=True)
    rows = jnp.sum(conf.T, axis=0, keepdims=True)
    cols = jnp.sum(conf, axis=0, keepdims=True)
    denom = rows + cols - tp
    dice = (tp + EPS) / (denom + EPS)
    dice_loss = jnp.sum((1.0 - dice) * w_ref[...]) / N_CLASSES
    cce_loss = cce_ref[0, 0] / n_total
    total = cce_loss * CCE_W + dice_loss * DICE_W
    out_ref[...] = jnp.full((1, 1), total, dtype=jnp.float32)


def kernel(predictions, ground_truth, class_weights):
    n, c = predictions.shape
    xT = predictions.T
    gT = ground_truth.T
    w2 = class_weights.reshape(c, 1)
    grid = (n // BN,)
    cce, bins = pl.pallas_call(
        _tc_kernel,
        grid=grid,
        in_specs=[
            pl.BlockSpec((c, BN), lambda i: (0, i)),
            pl.BlockSpec((c, BN), lambda i: (0, i)),
            pl.BlockSpec((c, 1), lambda i: (0, 0)),
        ],
        out_specs=[
            pl.BlockSpec((1, 1), lambda i: (0, 0)),
            pl.BlockSpec((1, BN), lambda i: (0, i)),
        ],
        out_shape=[
            jax.ShapeDtypeStruct((1, 1), jnp.float32),
            jax.ShapeDtypeStruct((1, n), jnp.int32),
        ],
        scratch_shapes=[pltpu.SMEM((1, 1), jnp.float32)],
    )(xT, gT, w2)

    per_w = n // _NW
    ones_in = jnp.ones((_CHUNK, _L), jnp.float32)
    zeros_in = jnp.zeros((_NBINS, _L), jnp.float32)
    parts = _make_sc_hist(n)(bins.reshape(n), ones_in, zeros_in)

    out = pl.pallas_call(
        functools.partial(_combine_kernel, n_total=n),
        in_specs=[
            pl.BlockSpec((_NC, _NBINS, _L), lambda: (0, 0, 0)),
            pl.BlockSpec((1, 1), lambda: (0, 0)),
            pl.BlockSpec((1, c), lambda: (0, 0)),
        ],
        out_specs=pl.BlockSpec((1, 1), lambda: (0, 0)),
        out_shape=jax.ShapeDtypeStruct((1, 1), jnp.float32),
    )(parts, cce, class_weights.reshape(1, c))
    return out.reshape(())
